# baseline (speedup 1.0000x reference)
import functools

import jax
import jax.numpy as jnp
from jax import lax
from jax.experimental import pallas as pl
from jax.experimental.pallas import tpu as pltpu
from jax.experimental.pallas import tpu_sc as plsc

_DIM = 128
_NBATCH = 2
_V = _DIM * _DIM * _DIM
_N = _NBATCH * _V
_NS = 16
_PER_W = _V // _NS
_CH = 2048
_NG = _CH // 16
_GCH = _PER_W // _CH
_HALO = 16513
_PAD = 16528
_OFFS = (0, 1, 128, 129, 16384, 16385, 16512, 16513)

_mesh = plsc.VectorSubcoreMesh(
    core_axis_name="c", subcore_axis_name="s", num_cores=2, num_subcores=16
)
_params = pltpu.CompilerParams(
    needs_layout_passes=False, use_tc_tiling_on_sc=False)


@functools.partial(
    pl.kernel,
    out_type=(
        jax.ShapeDtypeStruct((_N, 8), jnp.float32),
        jax.ShapeDtypeStruct((_N,), jnp.float32),
    ),
    mesh=_mesh,
    scratch_types=[
        pltpu.VMEM((_CH + _PAD,), jnp.float32),
        pltpu.VMEM((_CH, 8), jnp.float32),
        pltpu.VMEM((3 * _CH,), jnp.float32),
        pltpu.VMEM((_CH,), jnp.int32),
        pltpu.VMEM((3, _CH), jnp.float32),
        pltpu.VMEM((_CH, 8), jnp.float32),
        pltpu.VMEM((_CH,), jnp.float32),
        pltpu.SemaphoreType.DMA,
    ],
    compiler_params=_params,
)
def _warp(ddf_hbm, img_hbm, tab_hbm, out_hbm,
          img_v, tab_v, ddf_v, idx_v, w_v, gat_v, out_v, sem):
    core = lax.axis_index("c")
    sub = lax.axis_index("s")
    tile_base = core * _V + sub * _PER_W
    iota = lax.iota(jnp.int32, 16)

    def build_chunk(g, _):
        base = tile_base + g * _CH
        pltpu.sync_copy(img_hbm.at[pl.ds(base, _CH + _PAD)], img_v)

        def group_body(i, _):
            o = i * 16
            rows = o + iota
            for c, off in enumerate(_OFFS):
                v = img_v[pl.ds(o + off, 16)]
                plsc.store_scatter(
                    tab_v, [rows, jnp.full((16,), c, jnp.int32)], v)
            return _

        lax.fori_loop(0, _NG, group_body, None)
        pltpu.sync_copy(tab_v, tab_hbm.at[pl.ds(base, _CH), :])
        return _

    lax.fori_loop(0, _GCH, build_chunk, None)

    plsc.subcore_barrier()

    def axis_split(coord_i, d, hi):
        loc = coord_i.astype(jnp.float32) + d
        loc = jnp.minimum(jnp.maximum(loc, 0.0), float(hi))
        f_i = jnp.minimum(loc.astype(jnp.int32), hi - 1)
        w = loc - f_i.astype(jnp.float32)
        return f_i, w

    def warp_chunk(g, _):
        base = tile_base + g * _CH
        pltpu.sync_copy(ddf_hbm.at[pl.ds(base * 3, 3 * _CH)], ddf_v)

        def idx_body(i, _):
            o = i * 16
            sl = pl.ds(o, 16)
            src = 3 * o + 3 * iota
            dx = plsc.load_gather(ddf_v, [src])
            dy = plsc.load_gather(ddf_v, [src + 1])
            dz = plsc.load_gather(ddf_v, [src + 2])
            n = base + o + iota
            ix, wx = axis_split((n >> 14) & 127, dx, 127)
            iy, wy = axis_split((n >> 7) & 127, dy, 127)
            iz, wz = axis_split(n & 127, dz, 127)
            idx_v[sl] = ((n >> 21) << 21) + (ix << 14) + (iy << 7) + iz
            w_v[0, sl] = wx
            w_v[1, sl] = wy
            w_v[2, sl] = wz
            return _

        lax.fori_loop(0, _NG, idx_body, None)

        pltpu.async_copy(tab_hbm.at[idx_v], gat_v, sem).wait()

        def blend_body(i, _):
            o = i * 16
            sl = pl.ds(o, 16)
            wx = w_v[0, sl]
            wy = w_v[1, sl]
            wz = w_v[2, sl]
            row = o + iota

            def corner(c):
                return plsc.load_gather(
                    gat_v, [row, jnp.full((16,), c, jnp.int32)])

            c00 = corner(0) * (1.0 - wz) + corner(1) * wz
            c01 = corner(2) * (1.0 - wz) + corner(3) * wz
            c10 = corner(4) * (1.0 - wz) + corner(5) * wz
            c11 = corner(6) * (1.0 - wz) + corner(7) * wz
            c0 = c00 * (1.0 - wy) + c01 * wy
            c1 = c10 * (1.0 - wy) + c11 * wy
            out_v[sl] = c0 * (1.0 - wx) + c1 * wx
            return _

        lax.fori_loop(0, _NG, blend_body, None)
        pltpu.sync_copy(out_v, out_hbm.at[pl.ds(base, _CH)])
        return _

    lax.fori_loop(0, _GCH, warp_chunk, None)


def kernel(ddf, image):
    img_flat = image.reshape(-1)
    img_pad = jnp.concatenate(
        [img_flat, jnp.zeros((_PAD,), dtype=img_flat.dtype)])
    _, out_flat = _warp(ddf.reshape(-1), img_pad)
    return out_flat.reshape(image.shape)


# speedup vs baseline: 6.1431x; 6.1431x over previous
import functools

import jax
import jax.numpy as jnp
from jax import lax
from jax.experimental import pallas as pl
from jax.experimental.pallas import tpu as pltpu
from jax.experimental.pallas import tpu_sc as plsc

_DIM = 128
_NBATCH = 2
_V = _DIM * _DIM * _DIM
_N = _NBATCH * _V
_NS = 16
_PER_W = _V // _NS
_CH = 2048
_NG = _CH // 16
_GCH = _PER_W // _CH
_HALO = 16513
_PAD = 16528
_OFFS = (0, 1, 128, 129, 16384, 16385, 16512, 16513)

_mesh = plsc.VectorSubcoreMesh(
    core_axis_name="c", subcore_axis_name="s", num_cores=2, num_subcores=16
)
_params = pltpu.CompilerParams(
    needs_layout_passes=False, use_tc_tiling_on_sc=False)


@functools.partial(
    pl.kernel,
    out_type=(
        jax.ShapeDtypeStruct((_N, 8), jnp.float32),
        jax.ShapeDtypeStruct((_N,), jnp.float32),
    ),
    mesh=_mesh,
    scratch_types=[
        pltpu.VMEM((_CH + _PAD,), jnp.float32),
        pltpu.VMEM((_CH, 8), jnp.float32),
        pltpu.VMEM((3 * _CH,), jnp.float32),
        pltpu.VMEM((_CH,), jnp.int32),
        pltpu.VMEM((3, _CH), jnp.float32),
        pltpu.VMEM((_CH, 8), jnp.float32),
        pltpu.VMEM((_CH,), jnp.float32),
        pltpu.SemaphoreType.DMA,
    ],
    compiler_params=_params,
)
def _warp(ddf_hbm, img_hbm, tab_hbm, out_hbm,
          img_v, tab_v, ddf_v, idx_v, w_v, gat_v, out_v, sem):
    core = lax.axis_index("c")
    sub = lax.axis_index("s")
    tile_base = core * _V + sub * _PER_W
    iota = lax.iota(jnp.int32, 16)

    def build_chunk(g, _):
        base = tile_base + g * _CH
        pltpu.sync_copy(img_hbm.at[pl.ds(base, _CH + _PAD)], img_v)

        def group_body(i, _):
            o = i * 16
            rows = o + iota
            for c, off in enumerate(_OFFS):
                v = img_v[pl.ds(o + off, 16)]
                plsc.store_scatter(
                    tab_v, [rows, jnp.full((16,), c, jnp.int32)], v)
            return _

        lax.fori_loop(0, _NG, group_body, None)
        pltpu.sync_copy(tab_v, tab_hbm.at[pl.ds(base, _CH), :])
        return _

    lax.fori_loop(0, _GCH, build_chunk, None)

    plsc.subcore_barrier()

    def axis_split(coord_i, d, hi):
        loc = coord_i.astype(jnp.float32) + d
        loc = jnp.minimum(jnp.maximum(loc, 0.0), float(hi))
        f_i = jnp.minimum(loc.astype(jnp.int32), hi - 1)
        w = loc - f_i.astype(jnp.float32)
        return f_i, w

    def warp_chunk(g, _):
        base = tile_base + g * _CH
        pltpu.sync_copy(ddf_hbm.at[pl.ds(base * 3, 3 * _CH)], ddf_v)

        def idx_body(i, _):
            o = i * 16
            sl = pl.ds(o, 16)
            dbase = (o >> 7) * 384 + (o & 127)
            dx = ddf_v[pl.ds(dbase, 16)]
            dy = ddf_v[pl.ds(dbase + 128, 16)]
            dz = ddf_v[pl.ds(dbase + 256, 16)]
            n = base + o + iota
            ix, wx = axis_split((n >> 14) & 127, dx, 127)
            iy, wy = axis_split((n >> 7) & 127, dy, 127)
            iz, wz = axis_split(n & 127, dz, 127)
            idx_v[sl] = ((n >> 21) << 21) + (ix << 14) + (iy << 7) + iz
            w_v[0, sl] = wx
            w_v[1, sl] = wy
            w_v[2, sl] = wz
            return _

        lax.fori_loop(0, _NG, idx_body, None)

        pltpu.async_copy(tab_hbm.at[idx_v], gat_v, sem).wait()

        def blend_body(i, _):
            o = i * 16
            sl = pl.ds(o, 16)
            wx = w_v[0, sl]
            wy = w_v[1, sl]
            wz = w_v[2, sl]
            row = o + iota

            def corner(c):
                return plsc.load_gather(
                    gat_v, [row, jnp.full((16,), c, jnp.int32)])

            c00 = corner(0) * (1.0 - wz) + corner(1) * wz
            c01 = corner(2) * (1.0 - wz) + corner(3) * wz
            c10 = corner(4) * (1.0 - wz) + corner(5) * wz
            c11 = corner(6) * (1.0 - wz) + corner(7) * wz
            c0 = c00 * (1.0 - wy) + c01 * wy
            c1 = c10 * (1.0 - wy) + c11 * wy
            out_v[sl] = c0 * (1.0 - wx) + c1 * wx
            return _

        lax.fori_loop(0, _NG, blend_body, None)
        pltpu.sync_copy(out_v, out_hbm.at[pl.ds(base, _CH)])
        return _

    lax.fori_loop(0, _GCH, warp_chunk, None)


def kernel(ddf, image):
    img_flat = image.reshape(-1)
    img_pad = jnp.concatenate(
        [img_flat, jnp.zeros((_PAD,), dtype=img_flat.dtype)])
    ddf_t = jnp.transpose(ddf, (0, 1, 2, 4, 3)).reshape(-1)
    _, out_flat = _warp(ddf_t, img_pad)
    return out_flat.reshape(image.shape)


# speedup vs baseline: 8.8321x; 1.4377x over previous
import functools

import jax
import jax.numpy as jnp
from jax import lax
from jax.experimental import pallas as pl
from jax.experimental.pallas import tpu as pltpu
from jax.experimental.pallas import tpu_sc as plsc

_DIM = 128
_NBATCH = 2
_V = _DIM * _DIM * _DIM
_N = _NBATCH * _V
_NS = 16
_PER_W = _V // _NS
_CH = 2048
_NG = _CH // 16
_GCH = _PER_W // _CH
_HALO = 16513
_PAD = 16528
_OFFS = (0, 1, 128, 129, 16384, 16385, 16512, 16513)

_mesh = plsc.VectorSubcoreMesh(
    core_axis_name="c", subcore_axis_name="s", num_cores=2, num_subcores=16
)
_params = pltpu.CompilerParams(
    needs_layout_passes=False, use_tc_tiling_on_sc=False)


@functools.partial(
    pl.kernel,
    out_type=(
        jax.ShapeDtypeStruct((_N, 8), jnp.float32),
        jax.ShapeDtypeStruct((_N,), jnp.float32),
    ),
    mesh=_mesh,
    scratch_types=[
        pltpu.VMEM((_CH + _PAD,), jnp.float32),
        pltpu.VMEM((_CH, 8), jnp.float32),
        pltpu.VMEM((3 * _CH,), jnp.float32),
        pltpu.VMEM((_CH,), jnp.int32),
        pltpu.VMEM((3, _CH), jnp.float32),
        pltpu.VMEM((_CH, 8), jnp.float32),
        pltpu.VMEM((_CH,), jnp.float32),
        pltpu.SemaphoreType.DMA,
    ],
    compiler_params=_params,
)
def _warp(ddf_hbm, img_hbm, tab_hbm, out_hbm,
          img_v, tab_v, ddf_v, idx_v, w_v, gat_v, out_v, sem):
    core = lax.axis_index("c")
    sub = lax.axis_index("s")
    tile_base = core * _V + sub * _PER_W
    iota = lax.iota(jnp.int32, 16)

    def build_chunk(g, _):
        base = tile_base + g * _CH
        pltpu.sync_copy(img_hbm.at[pl.ds(base, _CH + _PAD)], img_v)

        @plsc.parallel_loop(0, _NG, unroll=4)
        def group_body(i):
            o = i * 16
            rows = o + iota
            for c, off in enumerate(_OFFS):
                v = img_v[pl.ds(o + off, 16)]
                plsc.store_scatter(
                    tab_v, [rows, jnp.full((16,), c, jnp.int32)], v)

        pltpu.sync_copy(tab_v, tab_hbm.at[pl.ds(base, _CH), :])
        return _

    lax.fori_loop(0, _GCH, build_chunk, None)

    plsc.subcore_barrier()

    def axis_split(coord_i, d, hi):
        loc = coord_i.astype(jnp.float32) + d
        loc = jnp.minimum(jnp.maximum(loc, 0.0), float(hi))
        f_i = jnp.minimum(loc.astype(jnp.int32), hi - 1)
        w = loc - f_i.astype(jnp.float32)
        return f_i, w

    batch_base = core << 21

    def warp_chunk(g, _):
        base = tile_base + g * _CH
        pltpu.sync_copy(ddf_hbm.at[pl.ds(base * 3, 3 * _CH)], ddf_v)

        @plsc.parallel_loop(0, _NG, unroll=4)
        def idx_body(i):
            o = i * 16
            sl = pl.ds(o, 16)
            dbase = (o >> 7) * 384 + (o & 127)
            dx = ddf_v[pl.ds(dbase, 16)]
            dy = ddf_v[pl.ds(dbase + 128, 16)]
            dz = ddf_v[pl.ds(dbase + 256, 16)]
            row = base + o
            ix, wx = axis_split((row >> 14) & 127, dx, 127)
            iy, wy = axis_split((row >> 7) & 127, dy, 127)
            iz, wz = axis_split((o & 127) + iota, dz, 127)
            idx_v[sl] = (batch_base + (ix << 14)) + ((iy << 7) + iz)
            w_v[0, sl] = wx
            w_v[1, sl] = wy
            w_v[2, sl] = wz

        pltpu.async_copy(tab_hbm.at[idx_v], gat_v, sem).wait()

        @plsc.parallel_loop(0, _NG, unroll=4)
        def blend_body(i):
            o = i * 16
            sl = pl.ds(o, 16)
            wx = w_v[0, sl]
            wy = w_v[1, sl]
            wz = w_v[2, sl]
            row = o + iota

            def corner(c):
                return plsc.load_gather(
                    gat_v, [row, jnp.full((16,), c, jnp.int32)])

            c00 = corner(0)
            c00 += wz * (corner(1) - c00)
            c01 = corner(2)
            c01 += wz * (corner(3) - c01)
            c10 = corner(4)
            c10 += wz * (corner(5) - c10)
            c11 = corner(6)
            c11 += wz * (corner(7) - c11)
            c0 = c00 + wy * (c01 - c00)
            c1 = c10 + wy * (c11 - c10)
            out_v[sl] = c0 + wx * (c1 - c0)

        pltpu.sync_copy(out_v, out_hbm.at[pl.ds(base, _CH)])
        return _

    lax.fori_loop(0, _GCH, warp_chunk, None)


def kernel(ddf, image):
    img_flat = image.reshape(-1)
    img_pad = jnp.concatenate(
        [img_flat, jnp.zeros((_PAD,), dtype=img_flat.dtype)])
    ddf_t = jnp.transpose(ddf, (0, 1, 2, 4, 3)).reshape(-1)
    _, out_flat = _warp(ddf_t, img_pad)
    return out_flat.reshape(image.shape)
